# Initial kernel scaffold; baseline (speedup 1.0000x reference)
import jax
import jax.numpy as jnp
from jax.experimental import pallas as pl


def kernel(output, anchors, targets):
    raise NotImplementedError("write your pallas kernel here")


# speedup vs baseline: 6.9380x; 6.9380x over previous
import functools

import jax
import jax.numpy as jnp
from jax import lax
from jax.experimental import pallas as pl
from jax.experimental.pallas import tpu as pltpu
from jax.experimental.pallas import tpu_sc as plsc

_B, _A, _H, _W = 32, 5, 26, 26
_NT = 50
_NTP = 64
_FLAT = _A * _H * _W * 5
_FPAD = 16912
_TPAD = 256
_THRESH = 0.5


def _rsqrt(v):
    i = plsc.bitcast(v, jnp.int32)
    y = plsc.bitcast(jnp.int32(0x5F3759DF) - (i >> 1), jnp.float32)
    for _ in range(3):
        y = y * (1.5 - 0.5 * v * y * y)
    return jnp.where(v == 0.0, jnp.float32(jnp.inf), y)


def _body(out_hbm, anc_hbm, tg_hbm, res_hbm,
          outbuf, tbuf, anc_v, cells, keptv, flagv, basev,
          g0r, g1r, g2r, g3r, table, accv, sumbuf, shared):
    cid = lax.axis_index("c")
    sid = lax.axis_index("s")
    b = cid * 16 + sid

    pltpu.sync_copy(out_hbm.at[pl.ds(b * _FPAD, _FPAD)], outbuf)
    pltpu.sync_copy(tg_hbm.at[pl.ds(b * _TPAD, _TPAD)], tbuf)
    pltpu.sync_copy(anc_hbm, anc_v)

    lane = lax.iota(jnp.int32, 16)
    av = anc_v[...]

    for ci in range(4):
        tvec = lane + 16 * ci
        valid = tvec < _NT
        trow = jnp.minimum(tvec, _NT - 1) * 5
        x = plsc.load_gather(tbuf, [trow + 1])
        y = plsc.load_gather(tbuf, [trow + 2])
        w = plsc.load_gather(tbuf, [trow + 3])
        h = plsc.load_gather(tbuf, [trow + 4])
        x = jnp.where(valid, x, 0.0)
        y = jnp.where(valid, y, 0.0)
        w = jnp.where(valid, w, 0.0)
        h = jnp.where(valid, h, 0.0)
        kept = valid & ~((x == 0.0) & (y == 0.0) & (w == 0.0) & (h == 0.0))

        cxf = x * float(_W)
        cyf = y * float(_H)
        cx = cxf.astype(jnp.int32)
        cy = cyf.astype(jnp.int32)
        ctx = cxf - cx.astype(jnp.float32) - 0.5
        cty = cyf - cy.astype(jnp.float32) - 0.5
        tw = w * float(_W)
        th = h * float(_H)
        t_area = tw * th

        best = jnp.full((16,), -1.0, jnp.float32)
        bidx = jnp.zeros((16,), jnp.int32)
        tx0 = ctx - tw * 0.5
        tx1 = ctx + tw * 0.5
        ty0 = cty - th * 0.5
        ty1 = cty + th * 0.5
        for a in range(_A):
            aw = av[2 * a]
            ah = av[2 * a + 1]
            aw2 = aw * 0.5
            ah2 = ah * 0.5
            x0 = jnp.maximum(tx0, -aw2)
            x1 = jnp.minimum(tx1, aw2)
            y0 = jnp.maximum(ty0, -ah2)
            y1 = jnp.minimum(ty1, ah2)
            ivl = (x0 < x1) & (y0 < y1)
            inter = jnp.where(ivl, (x1 - x0) * (y1 - y0), 0.0)
            iou = inter / (t_area + aw * ah - inter)
            upd = iou > best
            best = jnp.where(upd, iou, best)
            bidx = jnp.where(upd, jnp.int32(a), bidx)

        flagged = kept & (best > _THRESH)
        cell = cy * _W + cx
        sl = pl.ds(16 * ci, 16)
        cells[sl] = cell
        keptv[sl] = kept.astype(jnp.int32)
        flagv[sl] = flagged.astype(jnp.int32)
        basev[sl] = (cell * _A + bidx) * 5
        g0r[sl] = cxf
        g1r[sl] = cyf
        g2r[sl] = tw
        g3r[sl] = th

    for ci in range(4):
        sl = pl.ds(16 * ci, 16)
        cvec = cells[sl]
        kvec = keptv[sl]
        addrs = jnp.where(kvec > 0, cvec, jnp.int32(_H * _W))
        tvec = lane + 16 * ci
        nt_here = min(16, _NT - 16 * ci)
        for j in range(nt_here):
            plsc.store_scatter(table, [addrs], tvec, mask=lane == j)

    contrib = jnp.zeros((16,), jnp.float32)
    cnt = jnp.zeros((16,), jnp.float32)
    for ci in range(4):
        sl = pl.ds(16 * ci, 16)
        tvec = lane + 16 * ci
        cell = cells[sl]
        bse = basev[sl]
        winner = plsc.load_gather(table, [cell])
        wm = (flagv[sl] > 0) & (winner == tvec)
        p0 = plsc.load_gather(outbuf, [bse])
        p1 = plsc.load_gather(outbuf, [bse + 1])
        p2 = plsc.load_gather(outbuf, [bse + 2])
        p3 = plsc.load_gather(outbuf, [bse + 3])
        d0 = p0 - g0r[sl]
        d1 = p1 - g1r[sl]
        d2 = _rsqrt(p2) - _rsqrt(g2r[sl])
        d3 = _rsqrt(p3) - _rsqrt(g3r[sl])
        ssq = d0 * d0 + d1 * d1 + d2 * d2 + d3 * d3
        contrib = contrib + jnp.where(wm, ssq, 0.0)
        cnt = cnt + jnp.where(wm, 1.0, 0.0)

    n2 = jnp.sum(cnt)
    s = jnp.sum(contrib)
    n2v = jnp.zeros((16,), jnp.float32) + n2
    sv = jnp.where(lane == 0, s, 0.0)
    accv[...] = jnp.where(n2v > 0.0, sv / (2.0 * n2v), 0.0)
    pltpu.sync_copy(accv, shared.at[pl.ds(sid * 16, 16)])
    plsc.subcore_barrier()

    @pl.when(sid == 0)
    def _():
        pltpu.sync_copy(shared, sumbuf)
        acc = sumbuf[pl.ds(0, 16)]
        for i in range(1, 16):
            acc = acc + sumbuf[pl.ds(16 * i, 16)]
        accv[...] = acc
        pltpu.sync_copy(accv, res_hbm.at[cid])


def kernel(output, anchors, targets):
    out1d = jnp.pad(output.reshape(_B, _FLAT),
                    ((0, 0), (0, _FPAD - _FLAT))).reshape(-1)
    tg1d = jnp.pad(targets.reshape(_B, _NT * 5),
                   ((0, 0), (0, _TPAD - _NT * 5))).reshape(-1)
    anc_pad = jnp.zeros((16,), jnp.float32).at[: 2 * _A].set(anchors.reshape(-1))
    mesh = plsc.VectorSubcoreMesh(core_axis_name="c", subcore_axis_name="s")
    k = pl.kernel(
        _body,
        mesh=mesh,
        compiler_params=pltpu.CompilerParams(needs_layout_passes=False),
        out_type=jax.ShapeDtypeStruct((2, 16), jnp.float32),
        scratch_types=[
            pltpu.VMEM((_FPAD,), jnp.float32),
            pltpu.VMEM((_TPAD,), jnp.float32),
            pltpu.VMEM((16,), jnp.float32),
            pltpu.VMEM((_NTP,), jnp.int32),
            pltpu.VMEM((_NTP,), jnp.int32),
            pltpu.VMEM((_NTP,), jnp.int32),
            pltpu.VMEM((_NTP,), jnp.int32),
            pltpu.VMEM((_NTP,), jnp.float32),
            pltpu.VMEM((_NTP,), jnp.float32),
            pltpu.VMEM((_NTP,), jnp.float32),
            pltpu.VMEM((_NTP,), jnp.float32),
            pltpu.VMEM((_H * _W + 8,), jnp.int32),
            pltpu.VMEM((16,), jnp.float32),
            pltpu.VMEM((256,), jnp.float32),
            pltpu.VMEM_SHARED((256,), jnp.float32),
        ],
    )
    res = k(out1d, anc_pad, tg1d)
    return (res[0, 0] + res[1, 0]) / jnp.float32(_B)
